# baseline (speedup 1.0000x reference)
import jax
import jax.numpy as jnp
from jax.experimental import pallas as pl

_BATCH = 16384
_NF = 100
_D = 32
_BBLK = 256


def _body(c_ref, o_ref):
    o_ref[...] = jnp.broadcast_to(c_ref[...][None], (_BBLK, _NF, 2 * _D))


@jax.jit
def kernel(x_hat, mask, Wv, bv, missing_table, present_table):
    grid = (_BATCH // _BBLK,)
    return pl.pallas_call(
        _body,
        grid=grid,
        in_specs=[pl.BlockSpec((_NF, 2 * _D), lambda i: (0, 0))],
        out_specs=pl.BlockSpec((_BBLK, _NF, 2 * _D), lambda i: (i, 0, 0)),
        out_shape=jax.ShapeDtypeStruct((_BATCH, _NF, 2 * _D), jnp.float32),
    )(jnp.concatenate([missing_table, present_table], axis=1))


# speedup vs baseline: 4.5275x; 4.5275x over previous
import jax
import jax.numpy as jnp
from jax.experimental import pallas as pl

_BATCH = 16384
_NF = 100
_D = 32
_JBLK = 2


def _body(x_ref, m_ref, a_ref, b_ref, c_ref, o_ref):
    v = 1.0 - m_ref[...]
    u = x_ref[...] * v
    shape = (_JBLK, 2 * _D, _BATCH)
    ub = jnp.broadcast_to(u, shape)
    vb = jnp.broadcast_to(v, shape)
    ab = jnp.broadcast_to(a_ref[...], shape)
    bb = jnp.broadcast_to(b_ref[...], shape)
    cb = jnp.broadcast_to(c_ref[...], shape)
    o_ref[...] = ub * ab + (vb * bb + cb)


@jax.jit
def kernel(x_hat, mask, Wv, bv, missing_table, present_table):
    wv = Wv[:, 0]
    a_t = jnp.concatenate(
        [jnp.broadcast_to(wv, (_NF, _D)), jnp.zeros((_NF, _D), jnp.float32)],
        axis=1,
    ).reshape(_NF, 2 * _D, 1)
    b_t = jnp.concatenate(
        [jnp.broadcast_to(bv, (_NF, _D)), present_table - missing_table],
        axis=1,
    ).reshape(_NF, 2 * _D, 1)
    c_t = jnp.concatenate(
        [jnp.zeros((_NF, _D), jnp.float32), missing_table], axis=1
    ).reshape(_NF, 2 * _D, 1)
    x_t = x_hat.T.reshape(_NF, 1, _BATCH)
    m_t = mask.T.reshape(_NF, 1, _BATCH)
    grid = (_NF // _JBLK,)
    out_t = pl.pallas_call(
        _body,
        grid=grid,
        in_specs=[
            pl.BlockSpec((_JBLK, 1, _BATCH), lambda i: (i, 0, 0)),
            pl.BlockSpec((_JBLK, 1, _BATCH), lambda i: (i, 0, 0)),
            pl.BlockSpec((_JBLK, 2 * _D, 1), lambda i: (i, 0, 0)),
            pl.BlockSpec((_JBLK, 2 * _D, 1), lambda i: (i, 0, 0)),
            pl.BlockSpec((_JBLK, 2 * _D, 1), lambda i: (i, 0, 0)),
        ],
        out_specs=pl.BlockSpec((_JBLK, 2 * _D, _BATCH), lambda i: (i, 0, 0)),
        out_shape=jax.ShapeDtypeStruct((_NF, 2 * _D, _BATCH), jnp.float32),
    )(x_t, m_t, a_t, b_t, c_t)
    return jnp.transpose(out_t, (2, 0, 1))
